# baseline (speedup 1.0000x reference)
import functools

import jax
import jax.numpy as jnp
from jax import lax
from jax.experimental import pallas as pl
from jax.experimental.pallas import tpu as pltpu
from jax.experimental.pallas import tpu_sc as plsc

NC, NS = 2, 16
NW = NC * 8
B = 320000
D = 128
BPW = B // NW
C = 200
NCHUNK = BPW // C


def _gather_kernel(cell_hbm, hid_hbm, idx_hbm, out_cell, out_hid,
                   idx0, idx1, cell0, cell1, hid0, hid1,
                   g0, g1, w0, w1):
    sid = lax.axis_index("s")
    wid = sid * NC + lax.axis_index("c")
    base = wid * BPW
    bufs = ((idx0, cell0, hid0, g0, w0), (idx1, cell1, hid1, g1, w1))

    def off_of(chunk):
        return pl.multiple_of(base + chunk * C, 8)

    def fire(chunk, b):
        idx_v, cell_v, hid_v, gsem, _ = bufs[b]
        off = off_of(chunk)
        pltpu.sync_copy(idx_hbm.at[pl.ds(off, C)], idx_v)
        pltpu.async_copy(cell_hbm.at[idx_v], cell_v, gsem)
        pltpu.async_copy(hid_hbm.at[idx_v], hid_v, gsem)

    def gwait_wstart(chunk, b):
        idx_v, cell_v, hid_v, gsem, wsem = bufs[b]
        off = off_of(chunk)
        pltpu.make_async_copy(cell_hbm.at[idx_v], cell_v, gsem).wait()
        pltpu.make_async_copy(hid_hbm.at[idx_v], hid_v, gsem).wait()
        pltpu.async_copy(cell_v, out_cell.at[pl.ds(off, C)], wsem)
        pltpu.async_copy(hid_v, out_hid.at[pl.ds(off, C)], wsem)

    def wwait(chunk, b):
        _, cell_v, hid_v, _, wsem = bufs[b]
        off = off_of(chunk)
        pltpu.make_async_copy(cell_v, out_cell.at[pl.ds(off, C)], wsem).wait()
        pltpu.make_async_copy(hid_v, out_hid.at[pl.ds(off, C)], wsem).wait()

    @pl.when(sid < 8)
    def _():
        fire(0, 0)
        fire(1, 1)
        gwait_wstart(0, 0)

        @pl.loop(0, NCHUNK - 2, step=2)
        def _(g):
            wwait(g, 0)
            fire(g + 2, 0)
            gwait_wstart(g + 1, 1)
            wwait(g + 1, 1)
            fire(g + 3, 1)
            gwait_wstart(g + 2, 0)

        gwait_wstart(NCHUNK - 1, 1)
        wwait(NCHUNK - 2, 0)
        wwait(NCHUNK - 1, 1)


def kernel(prev_cell, prev_hidden, child_indices):
    mesh = plsc.VectorSubcoreMesh(core_axis_name="c", subcore_axis_name="s")
    run = functools.partial(
        pl.kernel,
        out_type=(
            jax.ShapeDtypeStruct((B, D), jnp.float32),
            jax.ShapeDtypeStruct((B, D), jnp.float32),
        ),
        mesh=mesh,
        scratch_types=[
            pltpu.VMEM((C,), jnp.int32),
            pltpu.VMEM((C,), jnp.int32),
            pltpu.VMEM((C, D), jnp.float32),
            pltpu.VMEM((C, D), jnp.float32),
            pltpu.VMEM((C, D), jnp.float32),
            pltpu.VMEM((C, D), jnp.float32),
            pltpu.SemaphoreType.DMA,
            pltpu.SemaphoreType.DMA,
            pltpu.SemaphoreType.DMA,
            pltpu.SemaphoreType.DMA,
        ],
    )(_gather_kernel)
    return run(prev_cell, prev_hidden, child_indices.astype(jnp.int32))


# speedup vs baseline: 1.7162x; 1.7162x over previous
import functools

import jax
import jax.numpy as jnp
from jax import lax
from jax.experimental import pallas as pl
from jax.experimental.pallas import tpu as pltpu
from jax.experimental.pallas import tpu_sc as plsc

NC, NS = 2, 16
B = 320000
D = 128
V = 10000
BPT = B // NS
C = 192
NCHUNK = 104
TAIL = BPT - NCHUNK * C
VSTAGE = 624


def _gather_kernel(cell_hbm, hid_hbm, idx_hbm, out_cell, out_hid,
                   table_sh, idx0, idx1, rows0, rows1,
                   g0, g1, w0, w1):
    cid = lax.axis_index("c")
    sid = lax.axis_index("s")
    base = sid * BPT
    bufs = ((idx0, rows0, g0, w0), (idx1, rows1, g1, w1))

    def run_table(table_hbm, out_hbm):
        voff = pl.multiple_of(sid * VSTAGE, 8)
        pltpu.sync_copy(table_hbm.at[pl.ds(voff, VSTAGE)],
                        table_sh.at[pl.ds(voff, VSTAGE)])

        @pl.when(sid == NS - 1)
        def _():
            rem = NS * VSTAGE
            pltpu.sync_copy(table_hbm.at[pl.ds(rem, V - rem)],
                            table_sh.at[pl.ds(rem, V - rem)])

        plsc.subcore_barrier()

        def fire(off, n, b):
            idx_v, rows_v, gsem, _ = bufs[b]
            pltpu.sync_copy(idx_hbm.at[pl.ds(off, n)],
                            idx_v.at[pl.ds(0, n)])
            pltpu.async_copy(table_sh.at[idx_v.at[pl.ds(0, n)]],
                             rows_v.at[pl.ds(0, n)], gsem)

        def gwait_wstart(off, n, b):
            idx_v, rows_v, gsem, wsem = bufs[b]
            pltpu.make_async_copy(table_sh.at[idx_v.at[pl.ds(0, n)]],
                                  rows_v.at[pl.ds(0, n)], gsem).wait()
            pltpu.async_copy(rows_v.at[pl.ds(0, n)],
                             out_hbm.at[pl.ds(off, n)], wsem)

        def wwait(off, n, b):
            _, rows_v, _, wsem = bufs[b]
            pltpu.make_async_copy(rows_v.at[pl.ds(0, n)],
                                  out_hbm.at[pl.ds(off, n)], wsem).wait()

        def off_of(chunk):
            return pl.multiple_of(base + chunk * C, 8)

        fire(off_of(0), C, 0)
        fire(off_of(1), C, 1)
        gwait_wstart(off_of(0), C, 0)

        @pl.loop(0, NCHUNK - 2, step=2)
        def _(g):
            wwait(off_of(g), C, 0)
            fire(off_of(g + 2), C, 0)
            gwait_wstart(off_of(g + 1), C, 1)
            wwait(off_of(g + 1), C, 1)
            fire(off_of(g + 3), C, 1)
            gwait_wstart(off_of(g + 2), C, 0)

        tail_off = pl.multiple_of(base + NCHUNK * C, 8)
        wwait(off_of(NCHUNK - 2), C, 0)
        fire(tail_off, TAIL, 0)
        gwait_wstart(off_of(NCHUNK - 1), C, 1)
        gwait_wstart(tail_off, TAIL, 0)
        wwait(off_of(NCHUNK - 1), C, 1)
        wwait(tail_off, TAIL, 0)

    @pl.when(cid == 0)
    def _():
        run_table(cell_hbm, out_cell)

    @pl.when(cid == 1)
    def _():
        run_table(hid_hbm, out_hid)


def kernel(prev_cell, prev_hidden, child_indices):
    mesh = plsc.VectorSubcoreMesh(core_axis_name="c", subcore_axis_name="s")
    run = functools.partial(
        pl.kernel,
        out_type=(
            jax.ShapeDtypeStruct((B, D), jnp.float32),
            jax.ShapeDtypeStruct((B, D), jnp.float32),
        ),
        mesh=mesh,
        scratch_types=[
            pltpu.VMEM_SHARED((V, D), jnp.float32),
            pltpu.VMEM((C,), jnp.int32),
            pltpu.VMEM((C,), jnp.int32),
            pltpu.VMEM((C, D), jnp.float32),
            pltpu.VMEM((C, D), jnp.float32),
            pltpu.SemaphoreType.DMA,
            pltpu.SemaphoreType.DMA,
            pltpu.SemaphoreType.DMA,
            pltpu.SemaphoreType.DMA,
        ],
    )(_gather_kernel)
    return run(prev_cell, prev_hidden, child_indices.astype(jnp.int32))


# speedup vs baseline: 2.2187x; 1.2928x over previous
import functools

import jax
import jax.numpy as jnp
from jax import lax
from jax.experimental import pallas as pl
from jax.experimental.pallas import tpu as pltpu
from jax.experimental.pallas import tpu_sc as plsc

NC, NS = 2, 16
B = 320000
D = 128
V = 10000
BPT = B // NS
C = 192
NCHUNK = 104
TAIL = BPT - NCHUNK * C
VSTAGE = 624


def _gather_kernel(cell_hbm, hid_hbm, idx_hbm, out_cell, out_hid,
                   table_sh, idx0, idx1, rows0, rows1,
                   g0, g1, w0, w1, i0, i1):
    cid = lax.axis_index("c")
    sid = lax.axis_index("s")
    base = sid * BPT
    bufs = ((idx0, rows0, g0, w0, i0), (idx1, rows1, g1, w1, i1))

    def run_table(table_hbm, out_hbm):
        voff = pl.multiple_of(sid * VSTAGE, 8)
        pltpu.sync_copy(table_hbm.at[pl.ds(voff, VSTAGE)],
                        table_sh.at[pl.ds(voff, VSTAGE)])

        @pl.when(sid == NS - 1)
        def _():
            rem = NS * VSTAGE
            pltpu.sync_copy(table_hbm.at[pl.ds(rem, V - rem)],
                            table_sh.at[pl.ds(rem, V - rem)])

        plsc.subcore_barrier()

        def pf(off, n, b):
            idx_v, _, _, _, isem = bufs[b]
            pltpu.async_copy(idx_hbm.at[pl.ds(off, n)],
                             idx_v.at[pl.ds(0, n)], isem)

        def fire(off, n, b):
            idx_v, rows_v, gsem, _, isem = bufs[b]
            pltpu.make_async_copy(idx_hbm.at[pl.ds(off, n)],
                                  idx_v.at[pl.ds(0, n)], isem).wait()
            pltpu.async_copy(table_sh.at[idx_v.at[pl.ds(0, n)]],
                             rows_v.at[pl.ds(0, n)], gsem)

        def gwait_wstart(off, n, b):
            idx_v, rows_v, gsem, wsem, _ = bufs[b]
            pltpu.make_async_copy(table_sh.at[idx_v.at[pl.ds(0, n)]],
                                  rows_v.at[pl.ds(0, n)], gsem).wait()
            pltpu.async_copy(rows_v.at[pl.ds(0, n)],
                             out_hbm.at[pl.ds(off, n)], wsem)

        def wwait(off, n, b):
            _, rows_v, _, wsem, _ = bufs[b]
            pltpu.make_async_copy(rows_v.at[pl.ds(0, n)],
                                  out_hbm.at[pl.ds(off, n)], wsem).wait()

        def off_of(chunk):
            return pl.multiple_of(base + chunk * C, 8)

        pf(off_of(0), C, 0)
        pf(off_of(1), C, 1)
        fire(off_of(0), C, 0)
        fire(off_of(1), C, 1)
        gwait_wstart(off_of(0), C, 0)
        pf(off_of(2), C, 0)

        @pl.loop(0, NCHUNK - 4, step=2)
        def _(g):
            wwait(off_of(g), C, 0)
            fire(off_of(g + 2), C, 0)
            gwait_wstart(off_of(g + 1), C, 1)
            pf(off_of(g + 3), C, 1)
            wwait(off_of(g + 1), C, 1)
            fire(off_of(g + 3), C, 1)
            gwait_wstart(off_of(g + 2), C, 0)
            pf(off_of(g + 4), C, 0)

        tail_off = pl.multiple_of(base + NCHUNK * C, 8)
        wwait(off_of(NCHUNK - 4), C, 0)
        fire(off_of(NCHUNK - 2), C, 0)
        gwait_wstart(off_of(NCHUNK - 3), C, 1)
        pf(off_of(NCHUNK - 1), C, 1)
        wwait(off_of(NCHUNK - 3), C, 1)
        fire(off_of(NCHUNK - 1), C, 1)
        gwait_wstart(off_of(NCHUNK - 2), C, 0)
        pf(tail_off, TAIL, 0)
        wwait(off_of(NCHUNK - 2), C, 0)
        fire(tail_off, TAIL, 0)
        gwait_wstart(off_of(NCHUNK - 1), C, 1)
        gwait_wstart(tail_off, TAIL, 0)
        wwait(off_of(NCHUNK - 1), C, 1)
        wwait(tail_off, TAIL, 0)

    @pl.when(cid == 0)
    def _():
        run_table(cell_hbm, out_cell)

    @pl.when(cid == 1)
    def _():
        run_table(hid_hbm, out_hid)


def kernel(prev_cell, prev_hidden, child_indices):
    mesh = plsc.VectorSubcoreMesh(core_axis_name="c", subcore_axis_name="s")
    run = functools.partial(
        pl.kernel,
        out_type=(
            jax.ShapeDtypeStruct((B, D), jnp.float32),
            jax.ShapeDtypeStruct((B, D), jnp.float32),
        ),
        mesh=mesh,
        scratch_types=[
            pltpu.VMEM_SHARED((V, D), jnp.float32),
            pltpu.VMEM((C,), jnp.int32),
            pltpu.VMEM((C,), jnp.int32),
            pltpu.VMEM((C, D), jnp.float32),
            pltpu.VMEM((C, D), jnp.float32),
            pltpu.SemaphoreType.DMA,
            pltpu.SemaphoreType.DMA,
            pltpu.SemaphoreType.DMA,
            pltpu.SemaphoreType.DMA,
            pltpu.SemaphoreType.DMA,
            pltpu.SemaphoreType.DMA,
        ],
    )(_gather_kernel)
    return run(prev_cell, prev_hidden, child_indices.astype(jnp.int32))
